# baseline (speedup 1.0000x reference)
import functools

import jax
import jax.numpy as jnp
from jax import lax
from jax.experimental import pallas as pl
from jax.experimental.pallas import tpu as pltpu
from jax.experimental.pallas import tpu_sc as plsc

B, L = 1024, 256
N = B * L
D = 50
DP = 56
NC, NS = 2, 16
NW = NC * NS
PW = N // NW
CH = 128
G = PW // CH
OW = CH * 2 * D
BPW = B // NW
HB = L // CH


def _gather_body(idxc_hbm, idxw_hbm, char_hbm, word_hbm, out_hbm,
                 idxc_v, idxw_v, bufc, bufw, bufp, semc, semw, semo):
    w = lax.axis_index("s") * NC + lax.axis_index("c")

    pltpu.sync_copy(idxc_hbm.at[w], idxc_v)
    pltpu.sync_copy(idxw_hbm.at[w], idxw_v)

    def fire(g, s):
        pltpu.async_copy(char_hbm.at[idxc_v.at[g]], bufc.at[s], semc)
        pltpu.async_copy(word_hbm.at[idxw_v.at[g]], bufw.at[s], semw)

    def wait_gather(s):
        pltpu.make_async_copy(char_hbm.at[idxc_v.at[0]], bufc.at[s], semc).wait()
        pltpu.make_async_copy(word_hbm.at[idxw_v.at[0]], bufw.at[s], semw).wait()

    def pack_and_out(g, s):
        def pack(p, carry2):
            for r8 in range(8):
                row = p * 8 + r8
                for off in (0, 16, 32, 34):
                    bufp[s, pl.ds(p * 800 + r8 * 100 + off, 16)] = bufc[s, row, pl.ds(off, 16)]
                    bufp[s, pl.ds(p * 800 + r8 * 100 + D + off, 16)] = bufw[s, row, pl.ds(off, 16)]
            return carry2

        lax.fori_loop(0, CH // 8, pack, 0)
        pltpu.async_copy(bufp.at[s], out_hbm.at[pl.ds((w * G + g) * OW, OW)], semo)

    def wait_out(g, s):
        pltpu.make_async_copy(bufp.at[s], out_hbm.at[pl.ds((w * G + g) * OW, OW)], semo).wait()

    fire(0, 0)

    def step(gg, carry):
        g0 = 2 * gg
        g1 = g0 + 1
        g2 = g0 + 2

        wait_gather(0)
        fire(g1, 1)

        @pl.when(gg >= 1)
        def _():
            wait_out(g0, 0)

        pack_and_out(g0, 0)

        @pl.when(gg < G // 2 - 1)
        def _():
            fire(g2, 0)

        wait_gather(1)

        @pl.when(gg >= 1)
        def _():
            wait_out(g1, 1)

        pack_and_out(g1, 1)
        return carry

    lax.fori_loop(0, G // 2, step, 0)
    wait_out(0, 0)
    wait_out(1, 1)


def _layout_body(emb_hbm, out_hbm, bufi, bufo, semi, semo):
    w = lax.axis_index("s") * NC + lax.axis_index("c")

    def fire(h, s):
        pltpu.async_copy(emb_hbm.at[pl.ds((w * G + h) * OW, OW)], bufi.at[s], semi)

    def wait_in(s):
        pltpu.make_async_copy(emb_hbm.at[pl.ds(0, OW)], bufi.at[s], semi).wait()

    def spread_and_out(h, s):
        for row in range(CH):
            for off in (0, 16, 32, 48, 64, 80, 84):
                bufo[s, row, pl.ds(off, 16)] = bufi[s, pl.ds(row * 100 + off, 16)]
        b = w * BPW + h // HB
        l0 = (h % HB) * CH
        cp = pltpu.async_copy(bufo.at[s], out_hbm.at[b].at[pl.ds(l0, CH), :], semo)
        return cp

    def wait_out(h, s):
        b = w * BPW + h // HB
        l0 = (h % HB) * CH
        pltpu.make_async_copy(bufo.at[s], out_hbm.at[b].at[pl.ds(l0, CH), :], semo).wait()

    fire(0, 0)

    def step(hh, carry):
        h0 = 2 * hh
        h1 = h0 + 1
        h2 = h0 + 2

        wait_in(0)
        fire(h1, 1)

        @pl.when(hh >= 1)
        def _():
            wait_out(h0, 0)

        spread_and_out(h0, 0)

        @pl.when(hh < G // 2 - 1)
        def _():
            fire(h2, 0)

        wait_in(1)

        @pl.when(hh >= 1)
        def _():
            wait_out(h1, 1)

        spread_and_out(h1, 1)
        return carry

    lax.fori_loop(0, G // 2, step, 0)
    wait_out(0, 0)
    wait_out(1, 1)


@jax.jit
def _lookup(idxc, idxw, char_table, word_table):
    mesh = plsc.VectorSubcoreMesh(core_axis_name="c", subcore_axis_name="s")
    gather = functools.partial(
        pl.kernel,
        mesh=mesh,
        out_type=jax.ShapeDtypeStruct((N * 2 * D,), jnp.float32),
        scratch_types=[
            pltpu.VMEM((G, CH), jnp.int32),
            pltpu.VMEM((G, CH), jnp.int32),
            pltpu.VMEM((2, CH, DP), jnp.float32),
            pltpu.VMEM((2, CH, DP), jnp.float32),
            pltpu.VMEM((2, OW), jnp.float32),
            pltpu.SemaphoreType.DMA,
            pltpu.SemaphoreType.DMA,
            pltpu.SemaphoreType.DMA,
        ],
        compiler_params=pltpu.CompilerParams(use_tc_tiling_on_sc=False),
    )(_gather_body)
    emb = gather(idxc, idxw, char_table, word_table)

    layout = functools.partial(
        pl.kernel,
        mesh=mesh,
        out_type=jax.ShapeDtypeStruct((B, L, 2 * D), jnp.float32),
        scratch_types=[
            pltpu.VMEM((2, OW), jnp.float32),
            pltpu.VMEM((2, CH, 2 * D), jnp.float32),
            pltpu.SemaphoreType.DMA,
            pltpu.SemaphoreType.DMA,
        ],
    )(_layout_body)
    return layout(emb)


def kernel(seqs_char, seqs_word, att_mask, char_table, word_table):
    idxc = seqs_char.astype(jnp.int32).reshape(NW, G, CH)
    idxw = seqs_word.astype(jnp.int32).reshape(NW, G, CH)
    ct = jnp.pad(char_table, ((0, 0), (0, DP - D)))
    wt = jnp.pad(word_table, ((0, 0), (0, DP - D)))
    return _lookup(idxc, idxw, ct, wt)


# speedup vs baseline: 1.2359x; 1.2359x over previous
import functools

import jax
import jax.numpy as jnp
from jax import lax
from jax.experimental import pallas as pl
from jax.experimental.pallas import tpu as pltpu
from jax.experimental.pallas import tpu_sc as plsc

B, L = 1024, 256
N = B * L
D = 50
DP = 64
WS = 2
NC, NS = 2, 16
NW = NC * NS
PW = N // NW
CH = 128
G = PW // CH
OW = CH * 128
BPW = B // NW
HB = L // CH

CHAR_COPIES = ((0, 0), (16, 16), (32, 32), (34, 34))
WORD_COPIES = ((2, 50), (18, 66), (34, 82), (36, 84))
SPREAD_OFFS = (0, 16, 32, 48, 64, 80, 84)


def _gather_body(idxc_hbm, idxw_hbm, char_hbm, word_hbm, out_hbm,
                 idxc_v, idxw_v, bufc, bufw, bufp, semc, semw, semo):
    w = lax.axis_index("s") * NC + lax.axis_index("c")

    pltpu.sync_copy(idxc_hbm.at[w], idxc_v)
    pltpu.sync_copy(idxw_hbm.at[w], idxw_v)

    def fire(g, s):
        pltpu.async_copy(char_hbm.at[idxc_v.at[g]], bufc.at[s], semc)
        pltpu.async_copy(word_hbm.at[idxw_v.at[g]], bufw.at[s], semw)

    def wait_gather(s):
        pltpu.make_async_copy(char_hbm.at[idxc_v.at[0]], bufc.at[s], semc).wait()
        pltpu.make_async_copy(word_hbm.at[idxw_v.at[0]], bufw.at[s], semw).wait()

    def pack_and_out(g, s):
        def pack(p, carry2):
            for r8 in range(8):
                row = p * 8 + r8
                dst = p * 1024 + r8 * 128
                for so, do in CHAR_COPIES:
                    bufp[s, pl.ds(dst + do, 16)] = bufc[s, row, pl.ds(so, 16)]
                for so, do in WORD_COPIES:
                    bufp[s, pl.ds(dst + do, 16)] = bufw[s, row, pl.ds(so, 16)]
            return carry2

        lax.fori_loop(0, CH // 8, pack, 0)
        pltpu.async_copy(bufp.at[s], out_hbm.at[pl.ds((w * G + g) * OW, OW)], semo)

    def wait_out(g, s):
        pltpu.make_async_copy(bufp.at[s], out_hbm.at[pl.ds((w * G + g) * OW, OW)], semo).wait()

    fire(0, 0)

    def step(gg, carry):
        g0 = 2 * gg
        g1 = g0 + 1
        g2 = g0 + 2

        wait_gather(0)
        fire(g1, 1)

        @pl.when(gg >= 1)
        def _():
            wait_out(g0, 0)

        pack_and_out(g0, 0)

        @pl.when(gg < G // 2 - 1)
        def _():
            fire(g2, 0)

        wait_gather(1)

        @pl.when(gg >= 1)
        def _():
            wait_out(g1, 1)

        pack_and_out(g1, 1)
        return carry

    lax.fori_loop(0, G // 2, step, 0)
    wait_out(0, 0)
    wait_out(1, 1)


def _layout_body(emb_hbm, out_hbm, bufi, bufo, semi, semo):
    w = lax.axis_index("s") * NC + lax.axis_index("c")

    def fire(h, s):
        pltpu.async_copy(emb_hbm.at[pl.ds((w * G + h) * OW, OW)], bufi.at[s], semi)

    def wait_in(s):
        pltpu.make_async_copy(emb_hbm.at[pl.ds(0, OW)], bufi.at[s], semi).wait()

    def spread_and_out(h, s):
        for row in range(CH):
            for off in SPREAD_OFFS:
                bufo[s, row, pl.ds(off, 16)] = bufi[s, pl.ds(row * 128 + off, 16)]
        b = w * BPW + h // HB
        l0 = (h % HB) * CH
        pltpu.async_copy(bufo.at[s], out_hbm.at[b].at[pl.ds(l0, CH), :], semo)

    def wait_out(h, s):
        b = w * BPW + h // HB
        l0 = (h % HB) * CH
        pltpu.make_async_copy(bufo.at[s], out_hbm.at[b].at[pl.ds(l0, CH), :], semo).wait()

    fire(0, 0)

    def step(hh, carry):
        h0 = 2 * hh
        h1 = h0 + 1
        h2 = h0 + 2

        wait_in(0)
        fire(h1, 1)

        @pl.when(hh >= 1)
        def _():
            wait_out(h0, 0)

        spread_and_out(h0, 0)

        @pl.when(hh < G // 2 - 1)
        def _():
            fire(h2, 0)

        wait_in(1)

        @pl.when(hh >= 1)
        def _():
            wait_out(h1, 1)

        spread_and_out(h1, 1)
        return carry

    lax.fori_loop(0, G // 2, step, 0)
    wait_out(0, 0)
    wait_out(1, 1)


@jax.jit
def _lookup(idxc, idxw, char_table, word_table):
    mesh = plsc.VectorSubcoreMesh(core_axis_name="c", subcore_axis_name="s")
    gather = functools.partial(
        pl.kernel,
        mesh=mesh,
        out_type=jax.ShapeDtypeStruct((N * 128,), jnp.float32),
        scratch_types=[
            pltpu.VMEM((G, CH), jnp.int32),
            pltpu.VMEM((G, CH), jnp.int32),
            pltpu.VMEM((2, CH, DP), jnp.float32),
            pltpu.VMEM((2, CH, DP), jnp.float32),
            pltpu.VMEM((2, OW), jnp.float32),
            pltpu.SemaphoreType.DMA,
            pltpu.SemaphoreType.DMA,
            pltpu.SemaphoreType.DMA,
        ],
        compiler_params=pltpu.CompilerParams(use_tc_tiling_on_sc=False),
    )(_gather_body)
    emb = gather(idxc, idxw, char_table, word_table)

    layout = functools.partial(
        pl.kernel,
        mesh=mesh,
        out_type=jax.ShapeDtypeStruct((B, L, 2 * D), jnp.float32),
        scratch_types=[
            pltpu.VMEM((2, OW), jnp.float32),
            pltpu.VMEM((2, CH, 2 * D), jnp.float32),
            pltpu.SemaphoreType.DMA,
            pltpu.SemaphoreType.DMA,
        ],
    )(_layout_body)
    return layout(emb)


def kernel(seqs_char, seqs_word, att_mask, char_table, word_table):
    idxc = seqs_char.astype(jnp.int32).reshape(NW, G, CH)
    idxw = seqs_word.astype(jnp.int32).reshape(NW, G, CH)
    ct = jnp.pad(char_table, ((0, 0), (0, DP - D)))
    wt = jnp.pad(word_table, ((0, 0), (WS, DP - D - WS)))
    return _lookup(idxc, idxw, ct, wt)


# speedup vs baseline: 1.6962x; 1.3724x over previous
import functools

import jax
import jax.numpy as jnp
from jax import lax
from jax.experimental import pallas as pl
from jax.experimental.pallas import tpu as pltpu
from jax.experimental.pallas import tpu_sc as plsc

B, L = 1024, 256
N = B * L
D = 50
DP = 64
WS = 2
NC, NS = 2, 16
NW = NC * NS
PW = N // NW
CH = 128
G = PW // CH
OW = CH * 128
NSLOT = 4

CHAR_COPIES = ((0, 0), (16, 16), (32, 32), (34, 34))
WORD_COPIES = ((2, 50), (18, 66), (34, 82), (36, 84))


def _gather_body(idxc_hbm, idxw_hbm, char_hbm, word_hbm, out_hbm,
                 idxc_v, idxw_v, bufc, bufw, bufp, semc, semw, semo):
    w = lax.axis_index("s") * NC + lax.axis_index("c")

    pltpu.sync_copy(idxc_hbm.at[w], idxc_v)
    pltpu.sync_copy(idxw_hbm.at[w], idxw_v)

    def fire(g, s):
        pltpu.async_copy(char_hbm.at[idxc_v.at[g]], bufc.at[s], semc)
        pltpu.async_copy(word_hbm.at[idxw_v.at[g]], bufw.at[s], semw)

    def wait_gather(s):
        pltpu.make_async_copy(char_hbm.at[idxc_v.at[0]], bufc.at[s], semc).wait()
        pltpu.make_async_copy(word_hbm.at[idxw_v.at[0]], bufw.at[s], semw).wait()

    def pack_and_out(g, s, j):
        def pack(p, carry2):
            for r8 in range(8):
                row = p * 8 + r8
                dst = p * 1024 + r8 * 128
                for so, do in CHAR_COPIES:
                    bufp[j, pl.ds(dst + do, 16)] = bufc[s, row, pl.ds(so, 16)]
                for so, do in WORD_COPIES:
                    bufp[j, pl.ds(dst + do, 16)] = bufw[s, row, pl.ds(so, 16)]
            return carry2

        lax.fori_loop(0, CH // 8, pack, 0)
        pltpu.async_copy(bufp.at[j], out_hbm.at[pl.ds((w * G + g) * OW, OW)], semo)

    def wait_out(g, j):
        pltpu.make_async_copy(bufp.at[j], out_hbm.at[pl.ds((w * G + g) * OW, OW)], semo).wait()

    for s in range(NSLOT):
        fire(s, s)

    def step(gg, carry):
        for k in range(NSLOT):
            g = NSLOT * gg + k
            j = k % 2

            wait_gather(k)

            if k >= 2:
                wait_out(g - 2, j)
            else:
                @pl.when(gg >= 1)
                def _():
                    wait_out(g - 2, j)

            pack_and_out(g, k, j)

            @pl.when(gg < G // NSLOT - 1)
            def _():
                fire(g + NSLOT, k)
        return carry

    lax.fori_loop(0, G // NSLOT, step, 0)
    wait_out(0, 0)
    wait_out(1, 1)


@jax.jit
def _lookup(idxc, idxw, char_table, word_table):
    mesh = plsc.VectorSubcoreMesh(core_axis_name="c", subcore_axis_name="s")
    gather = functools.partial(
        pl.kernel,
        mesh=mesh,
        out_type=jax.ShapeDtypeStruct((N * 128,), jnp.float32),
        scratch_types=[
            pltpu.VMEM((G, CH), jnp.int32),
            pltpu.VMEM((G, CH), jnp.int32),
            pltpu.VMEM((NSLOT, CH, DP), jnp.float32),
            pltpu.VMEM((NSLOT, CH, DP), jnp.float32),
            pltpu.VMEM((2, OW), jnp.float32),
            pltpu.SemaphoreType.DMA,
            pltpu.SemaphoreType.DMA,
            pltpu.SemaphoreType.DMA,
        ],
        compiler_params=pltpu.CompilerParams(use_tc_tiling_on_sc=False),
    )(_gather_body)
    return gather(idxc, idxw, char_table, word_table)


def kernel(seqs_char, seqs_word, att_mask, char_table, word_table):
    idxc = seqs_char.astype(jnp.int32).reshape(NW, G, CH)
    idxw = seqs_word.astype(jnp.int32).reshape(NW, G, CH)
    ct = jnp.pad(char_table, ((0, 0), (0, DP - D)))
    wt = jnp.pad(word_table, ((0, 0), (WS, DP - D - WS)))
    emb = _lookup(idxc, idxw, ct, wt)
    return emb.reshape(B, L, 128)[:, :, : 2 * D]


# speedup vs baseline: 1.7130x; 1.0099x over previous
import functools

import jax
import jax.numpy as jnp
from jax import lax
from jax.experimental import pallas as pl
from jax.experimental.pallas import tpu as pltpu
from jax.experimental.pallas import tpu_sc as plsc

B, L = 1024, 256
N = B * L
D = 50
DP = 64
WS = 2
NC, NS = 2, 16
NW = NC * NS
PW = N // NW
CH = 128
G = PW // CH
OW = CH * 128
BPW = B // NW
HB = L // CH

CHAR_COPIES = ((0, 0), (16, 16), (32, 32), (34, 34))
WORD_COPIES = ((2, 50), (18, 66), (34, 82), (36, 84))
SPREAD_OFFS = (0, 16, 32, 48, 64, 80, 84)


def _gather_body(idxc_hbm, idxw_hbm, char_hbm, word_hbm, out_hbm,
                 idxc_v, idxw_v, bufc, bufw, bufp, semc, semw, semo):
    w = lax.axis_index("s") * NC + lax.axis_index("c")

    pltpu.sync_copy(idxc_hbm.at[w], idxc_v)
    pltpu.sync_copy(idxw_hbm.at[w], idxw_v)

    def fire(g, s):
        pltpu.async_copy(char_hbm.at[idxc_v.at[g]], bufc.at[s], semc)
        pltpu.async_copy(word_hbm.at[idxw_v.at[g]], bufw.at[s], semw)

    def wait_gather(s):
        pltpu.make_async_copy(char_hbm.at[idxc_v.at[0]], bufc.at[s], semc).wait()
        pltpu.make_async_copy(word_hbm.at[idxw_v.at[0]], bufw.at[s], semw).wait()

    def pack_and_out(g, s):
        def pack(p, carry2):
            for r8 in range(8):
                row = p * 8 + r8
                dst = p * 1024 + r8 * 128
                for so, do in CHAR_COPIES:
                    bufp[s, pl.ds(dst + do, 16)] = bufc[s, row, pl.ds(so, 16)]
                for so, do in WORD_COPIES:
                    bufp[s, pl.ds(dst + do, 16)] = bufw[s, row, pl.ds(so, 16)]
            return carry2

        lax.fori_loop(0, CH // 8, pack, 0)
        pltpu.async_copy(bufp.at[s], out_hbm.at[pl.ds((w * G + g) * OW, OW)], semo)

    def wait_out(g, s):
        pltpu.make_async_copy(bufp.at[s], out_hbm.at[pl.ds((w * G + g) * OW, OW)], semo).wait()

    fire(0, 0)

    def step(gg, carry):
        g0 = 2 * gg
        g1 = g0 + 1
        g2 = g0 + 2

        wait_gather(0)
        fire(g1, 1)

        @pl.when(gg >= 1)
        def _():
            wait_out(g0, 0)

        pack_and_out(g0, 0)

        @pl.when(gg < G // 2 - 1)
        def _():
            fire(g2, 0)

        wait_gather(1)

        @pl.when(gg >= 1)
        def _():
            wait_out(g1, 1)

        pack_and_out(g1, 1)
        return carry

    lax.fori_loop(0, G // 2, step, 0)
    wait_out(0, 0)
    wait_out(1, 1)


def _layout_body(emb_hbm, out_hbm, bufi, bufo, semi, semo):
    w = lax.axis_index("s") * NC + lax.axis_index("c")

    def fire(h, s):
        pltpu.async_copy(emb_hbm.at[pl.ds((w * G + h) * OW, OW)], bufi.at[s], semi)

    def wait_in(s):
        pltpu.make_async_copy(emb_hbm.at[pl.ds(0, OW)], bufi.at[s], semi).wait()

    def spread_and_out(h, s):
        for row in range(CH):
            for off in SPREAD_OFFS:
                bufo[s, row, pl.ds(off, 16)] = bufi[s, pl.ds(row * 128 + off, 16)]
        b = w * BPW + h // HB
        l0 = (h % HB) * CH
        pltpu.async_copy(bufo.at[s], out_hbm.at[b].at[pl.ds(l0, CH), :], semo)

    def wait_out(h, s):
        b = w * BPW + h // HB
        l0 = (h % HB) * CH
        pltpu.make_async_copy(bufo.at[s], out_hbm.at[b].at[pl.ds(l0, CH), :], semo).wait()

    fire(0, 0)

    def step(hh, carry):
        h0 = 2 * hh
        h1 = h0 + 1
        h2 = h0 + 2

        wait_in(0)
        fire(h1, 1)

        @pl.when(hh >= 1)
        def _():
            wait_out(h0, 0)

        spread_and_out(h0, 0)

        @pl.when(hh < G // 2 - 1)
        def _():
            fire(h2, 0)

        wait_in(1)

        @pl.when(hh >= 1)
        def _():
            wait_out(h1, 1)

        spread_and_out(h1, 1)
        return carry

    lax.fori_loop(0, G // 2, step, 0)
    wait_out(0, 0)
    wait_out(1, 1)


@jax.jit
def _lookup(idxc, idxw, char_table, word_table):
    mesh = plsc.VectorSubcoreMesh(core_axis_name="c", subcore_axis_name="s")
    gather = functools.partial(
        pl.kernel,
        mesh=mesh,
        out_type=jax.ShapeDtypeStruct((N * 128,), jnp.float32),
        scratch_types=[
            pltpu.VMEM((G, CH), jnp.int32),
            pltpu.VMEM((G, CH), jnp.int32),
            pltpu.VMEM((2, CH, DP), jnp.float32),
            pltpu.VMEM((2, CH, DP), jnp.float32),
            pltpu.VMEM((2, OW), jnp.float32),
            pltpu.SemaphoreType.DMA,
            pltpu.SemaphoreType.DMA,
            pltpu.SemaphoreType.DMA,
        ],
        compiler_params=pltpu.CompilerParams(use_tc_tiling_on_sc=False),
    )(_gather_body)
    emb = gather(idxc, idxw, char_table, word_table)
    return emb.reshape(B, L, 128)[:, :, : 2 * D]


def kernel(seqs_char, seqs_word, att_mask, char_table, word_table):
    idxc = (seqs_char.astype(jnp.int32) % 7002).reshape(NW, G, CH)
    idxw = (seqs_word.astype(jnp.int32) % 100002).reshape(NW, G, CH)
    ct = jnp.pad(char_table, ((0, 0), (0, DP - D)))
    wt = jnp.pad(word_table, ((0, 0), (WS, DP - D - WS)))
    return _lookup(idxc, idxw, ct, wt)


# speedup vs baseline: 1.7131x; 1.0001x over previous
import functools

import jax
import jax.numpy as jnp
from jax import lax
from jax.experimental import pallas as pl
from jax.experimental.pallas import tpu as pltpu
from jax.experimental.pallas import tpu_sc as plsc

B, L = 1024, 256
N = B * L
D = 50
DP = 64
WS = 2
NC, NS = 2, 16
NW = NC * NS
PW = N // NW
CH = 128
G = PW // CH
OW = CH * 128

CHAR_COPIES = ((0, 0), (16, 16), (32, 32), (34, 34))
WORD_COPIES = ((2, 50), (18, 66), (34, 82), (36, 84))


def _gather_body(idxc_hbm, idxw_hbm, char_hbm, word_hbm, out_hbm,
                 idxc_v, idxw_v, bufc, bufw, bufp, semc, semw, semo):
    w = lax.axis_index("s") * NC + lax.axis_index("c")

    pltpu.sync_copy(idxc_hbm.at[w], idxc_v)
    pltpu.sync_copy(idxw_hbm.at[w], idxw_v)

    def fire(g, s):
        pltpu.async_copy(char_hbm.at[idxc_v.at[g]], bufc.at[s], semc)
        pltpu.async_copy(word_hbm.at[idxw_v.at[g]], bufw.at[s], semw)

    def wait_gather(s):
        pltpu.make_async_copy(char_hbm.at[idxc_v.at[0]], bufc.at[s], semc).wait()
        pltpu.make_async_copy(word_hbm.at[idxw_v.at[0]], bufw.at[s], semw).wait()

    def pack_and_out(g, s):
        def pack(p, carry2):
            for r8 in range(8):
                row = p * 8 + r8
                dst = p * 1024 + r8 * 128
                for so, do in CHAR_COPIES:
                    bufp[s, pl.ds(dst + do, 16)] = bufc[s, row, pl.ds(so, 16)]
                for so, do in WORD_COPIES:
                    bufp[s, pl.ds(dst + do, 16)] = bufw[s, row, pl.ds(so, 16)]
            return carry2

        lax.fori_loop(0, CH // 8, pack, 0)
        pltpu.async_copy(bufp.at[s], out_hbm.at[pl.ds((w * G + g) * OW, OW)], semo)

    def wait_out(g, s):
        pltpu.make_async_copy(bufp.at[s], out_hbm.at[pl.ds((w * G + g) * OW, OW)], semo).wait()

    fire(0, 0)

    def step(gg, carry):
        g0 = 2 * gg
        g1 = g0 + 1
        g2 = g0 + 2

        wait_gather(0)
        fire(g1, 1)

        @pl.when(gg >= 1)
        def _():
            wait_out(g0, 0)

        pack_and_out(g0, 0)

        @pl.when(gg < G // 2 - 1)
        def _():
            fire(g2, 0)

        wait_gather(1)

        @pl.when(gg >= 1)
        def _():
            wait_out(g1, 1)

        pack_and_out(g1, 1)
        return carry

    lax.fori_loop(0, G // 2, step, 0)
    wait_out(0, 0)
    wait_out(1, 1)


@jax.jit
def _lookup(idxc, idxw, char_table, word_table):
    mesh = plsc.VectorSubcoreMesh(core_axis_name="c", subcore_axis_name="s")
    gather = functools.partial(
        pl.kernel,
        mesh=mesh,
        out_type=jax.ShapeDtypeStruct((N * 128,), jnp.float32),
        scratch_types=[
            pltpu.VMEM((G, CH), jnp.int32),
            pltpu.VMEM((G, CH), jnp.int32),
            pltpu.VMEM((2, CH, DP), jnp.float32),
            pltpu.VMEM((2, CH, DP), jnp.float32),
            pltpu.VMEM((2, OW), jnp.float32),
            pltpu.SemaphoreType.DMA,
            pltpu.SemaphoreType.DMA,
            pltpu.SemaphoreType.DMA,
        ],
        compiler_params=pltpu.CompilerParams(use_tc_tiling_on_sc=False),
    )(_gather_body)
    emb = gather(idxc, idxw, char_table, word_table)
    return emb.reshape(B, L, 128)[:, :, : 2 * D]


def kernel(seqs_char, seqs_word, att_mask, char_table, word_table):
    idxc = seqs_char.astype(jnp.int32).reshape(NW, G, CH)
    idxw = seqs_word.astype(jnp.int32).reshape(NW, G, CH)
    ct = jnp.pad(char_table, ((0, 0), (0, DP - D)))
    wt = jnp.pad(word_table, ((0, 0), (WS, DP - D - WS)))
    return _lookup(idxc, idxw, ct, wt)
